# baseline (speedup 1.0000x reference)
import functools

import jax
import jax.numpy as jnp
from jax import lax
from jax.experimental import pallas as pl
from jax.experimental.pallas import tpu as pltpu
from jax.experimental.pallas import tpu_sc as plsc

N = 10000
E = 160000
D = 100
NUM_REL = 16
NB = 4

DP = 128
CNT_COL = 100
NW = 32
EPW = 5120
EP = NW * EPW
CHUNK = 256
NCHUNK = EPW // CHUNK
NP = 10240
ROWS_PER_TILE = NP // 16
ZROWS = 16


def _y_body(x_ref, bf_ref, att_ref, y_ref):
    xb = jnp.dot(x_ref[...], bf_ref[...], preferred_element_type=jnp.float32)
    for r in range(NUM_REL):
        acc = att_ref[r, 0] * xb[:, 0:DP]
        for b in range(1, NB):
            acc = acc + att_ref[r, b] * xb[:, b * DP:(b + 1) * DP]
        y_ref[:, r * DP:(r + 1) * DP] = acc


def _y_matmul(xp, basisflat, att, blk=400):
    grid = N // blk
    return pl.pallas_call(
        _y_body,
        grid=(grid,),
        in_specs=[
            pl.BlockSpec((blk, DP), lambda i: (i, 0)),
            pl.BlockSpec((DP, NB * DP), lambda i: (0, 0)),
            pl.BlockSpec(memory_space=pltpu.SMEM),
        ],
        out_specs=pl.BlockSpec((blk, NUM_REL * DP), lambda i: (i, 0)),
        out_shape=jax.ShapeDtypeStruct((N, NUM_REL * DP), jnp.float32),
    )(xp, basisflat, att)


def _epi_body(a0_ref, a1_ref, x_ref, root_ref, bias_ref, o_ref, *, relu):
    s = a0_ref[...] + a1_ref[...]
    lanes = lax.broadcasted_iota(jnp.int32, s.shape, 1)
    cnt = jnp.sum(jnp.where(lanes == CNT_COL, s, 0.0), axis=1, keepdims=True)
    denom = jnp.maximum(cnt, 1.0)
    msg = jnp.where(lanes < D, s, 0.0) / denom
    out = msg + jnp.dot(x_ref[...], root_ref[...],
                        preferred_element_type=jnp.float32) + bias_ref[...]
    if relu:
        out = jnp.maximum(out, 0.0)
    o_ref[...] = out


def _epilogue(a0, a1, xp, root_p, bias_p, relu, blk=400):
    grid = N // blk
    return pl.pallas_call(
        functools.partial(_epi_body, relu=relu),
        grid=(grid,),
        in_specs=[
            pl.BlockSpec((blk, DP), lambda i: (i, 0)),
            pl.BlockSpec((blk, DP), lambda i: (i, 0)),
            pl.BlockSpec((blk, DP), lambda i: (i, 0)),
            pl.BlockSpec((DP, DP), lambda i: (0, 0)),
            pl.BlockSpec((1, DP), lambda i: (0, 0)),
        ],
        out_specs=pl.BlockSpec((blk, DP), lambda i: (i, 0)),
        out_shape=jax.ShapeDtypeStruct((N, DP), jnp.float32),
    )(a0, a1, xp, root_p, bias_p)


def _mid_body(a0_ref, a1_ref, x_ref, root_ref, bias_ref, bf_ref, att_ref,
              h_ref, y_ref):
    s = a0_ref[...] + a1_ref[...]
    lanes = lax.broadcasted_iota(jnp.int32, s.shape, 1)
    cnt = jnp.sum(jnp.where(lanes == CNT_COL, s, 0.0), axis=1, keepdims=True)
    denom = jnp.maximum(cnt, 1.0)
    msg = jnp.where(lanes < D, s, 0.0) / denom
    h = msg + jnp.dot(x_ref[...], root_ref[...],
                      preferred_element_type=jnp.float32) + bias_ref[...]
    h = jnp.maximum(h, 0.0)
    h_ref[...] = h
    xb = jnp.dot(h, bf_ref[...], preferred_element_type=jnp.float32)
    for r in range(NUM_REL):
        acc = att_ref[r, 0] * xb[:, 0:DP]
        for b in range(1, NB):
            acc = acc + att_ref[r, b] * xb[:, b * DP:(b + 1) * DP]
        y_ref[:, r * DP:(r + 1) * DP] = acc


def _mid(a0, a1, xp, root_p, bias_p, basisflat2, att2, blk=400):
    grid = N // blk
    return pl.pallas_call(
        _mid_body,
        grid=(grid,),
        in_specs=[
            pl.BlockSpec((blk, DP), lambda i: (i, 0)),
            pl.BlockSpec((blk, DP), lambda i: (i, 0)),
            pl.BlockSpec((blk, DP), lambda i: (i, 0)),
            pl.BlockSpec((DP, DP), lambda i: (0, 0)),
            pl.BlockSpec((1, DP), lambda i: (0, 0)),
            pl.BlockSpec((DP, NB * DP), lambda i: (0, 0)),
            pl.BlockSpec(memory_space=pltpu.SMEM),
        ],
        out_specs=[
            pl.BlockSpec((blk, DP), lambda i: (i, 0)),
            pl.BlockSpec((blk, NUM_REL * DP), lambda i: (i, 0)),
        ],
        out_shape=[
            jax.ShapeDtypeStruct((N, DP), jnp.float32),
            jax.ShapeDtypeStruct((N, NUM_REL * DP), jnp.float32),
        ],
    )(a0, a1, xp, root_p, bias_p, basisflat2, att2)


def _sc_body(y_hbm, gidx_hbm, dst_hbm, nsp_hbm, out_hbm,
             gidx_v, dst_v, nsp_v, rows_v, acc_sh, sem, dsem):
    cid = lax.axis_index("c")
    sid = lax.axis_index("s")
    wid = sid * 2 + cid

    pltpu.sync_copy(gidx_hbm.at[wid], gidx_v)

    zero = jnp.zeros((16,), jnp.float32)
    for q in range(ZROWS):
        for p in range(DP // 16):
            rows_v[q, pl.ds(p * 16, 16)] = zero

    def _zero_step(z, carry):
        pltpu.sync_copy(
            rows_v.at[pl.ds(0, ZROWS)],
            acc_sh.at[pl.ds(sid * ROWS_PER_TILE + z * ZROWS, ZROWS)])
        return carry
    lax.fori_loop(0, ROWS_PER_TILE // ZROWS, _zero_step, 0)
    plsc.subcore_barrier()

    onehot = jnp.where(lax.iota(jnp.int32, 16) == (CNT_COL % 16), 1.0, 0.0)
    cnt_p = CNT_COL // 16

    def _chunk(c, carry):
        desc = pltpu.async_copy(y_hbm.at[gidx_v.at[pl.ds(c * CHUNK, CHUNK)]], rows_v, sem)
        ddesc = pltpu.async_copy(dst_hbm.at[wid, c], dst_v, dsem)
        pltpu.sync_copy(nsp_hbm.at[wid, pl.ds(c * (CHUNK // 8), CHUNK // 8)],
                        nsp_v)
        desc.wait()

        def _grp(g, carry2):
            for k in range(8):
                j = g * 8 + k
                nv = nsp_v[g, pl.ds(k * 16, 16)]
                for p in range(DP // 16):
                    r = rows_v[j, pl.ds(p * 16, 16)] * nv
                    if p == cnt_p:
                        r = r + onehot
                    rows_v[j, pl.ds(p * 16, 16)] = r
            return carry2
        lax.fori_loop(0, CHUNK // 8, _grp, 0)
        ddesc.wait()
        pltpu.sync_copy(rows_v, acc_sh.at[dst_v], add=True)
        return carry
    lax.fori_loop(0, NCHUNK, _chunk, 0)
    plsc.subcore_barrier()

    pltpu.sync_copy(acc_sh.at[pl.ds(sid * ROWS_PER_TILE, ROWS_PER_TILE)],
                    out_hbm.at[cid, pl.ds(sid * ROWS_PER_TILE, ROWS_PER_TILE)])


def _sc_scatter(y, gidx, dst, nsp):
    mesh = plsc.VectorSubcoreMesh(core_axis_name="c", subcore_axis_name="s")
    f = pl.kernel(
        _sc_body,
        out_type=jax.ShapeDtypeStruct((2, NP, DP), jnp.float32),
        mesh=mesh,
        scratch_types=[
            pltpu.VMEM((EPW,), jnp.int32),
            pltpu.VMEM((CHUNK,), jnp.int32),
            pltpu.VMEM((CHUNK // 8, DP), jnp.float32),
            pltpu.VMEM((CHUNK, DP), jnp.float32),
            pltpu.VMEM_SHARED((NP, DP), jnp.float32),
            pltpu.SemaphoreType.DMA,
            pltpu.SemaphoreType.DMA,
        ],
    )
    return f(y.reshape(N * NUM_REL, DP), gidx, dst, nsp)


def _two_layers(xp, gidx, dst, nsp, bf1, att1, rp1, bp1, bf2, att2, rp2, bp2):
    y1 = _y_matmul(xp, bf1, att1)
    acc1 = _sc_scatter(y1, gidx, dst, nsp)
    h, y2 = _mid(acc1[0], acc1[1], xp, rp1, bp1, bf2, att2)
    acc2 = _sc_scatter(y2, gidx, dst, nsp)
    return _epilogue(acc2[0], acc2[1], h, rp2, bp2, relu=False)


def kernel(entity, edge_index, edge_attr, edge_type, edge_norm,
           basis1, att1, root1, bias1, We1, be1,
           basis2, att2, root2, bias2, We2, be2):
    del edge_attr, We1, be1, We2, be2

    src = edge_index[0]
    dst = edge_index[1]
    et = edge_type[:, 0]
    pad = EP - E
    gidx = jnp.pad((src * NUM_REL + et).astype(jnp.int32), (0, pad))
    gidx = gidx.reshape(NW, EPW)
    dst_r = jnp.pad(dst.astype(jnp.int32), (0, pad), constant_values=N)
    dst_r = dst_r.reshape(NW, NCHUNK, CHUNK)
    nsp = jnp.pad(edge_norm, (0, pad))
    nsp = jnp.broadcast_to(nsp[:, None], (EP, 16)).reshape(NW, EPW // 8, DP)

    xp = jnp.pad(entity, ((0, 0), (0, DP - D)))

    def prep(basis, root, bias):
        bf = jnp.pad(jnp.transpose(basis, (1, 0, 2)),
                     ((0, DP - D), (0, 0), (0, DP - D))).reshape(DP, NB * DP)
        rp = jnp.pad(root, ((0, DP - D), (0, DP - D)))
        bp = jnp.pad(bias, (0, DP - D)).reshape(1, DP)
        return bf, rp, bp

    bf1, rp1, bp1 = prep(basis1, root1, bias1)
    bf2, rp2, bp2 = prep(basis2, root2, bias2)

    out = _two_layers(xp, gidx, dst_r, nsp, bf1, att1, rp1, bp1,
                      bf2, att2, rp2, bp2)
    return out[:, :D]


# speedup vs baseline: 1.0478x; 1.0478x over previous
import functools

import jax
import jax.numpy as jnp
from jax import lax
from jax.experimental import pallas as pl
from jax.experimental.pallas import tpu as pltpu
from jax.experimental.pallas import tpu_sc as plsc

N = 10000
E = 160000
D = 100
NUM_REL = 16
NB = 4

DP = 128
CNT_COL = 100
NW = 32
EPW = 5120
EP = NW * EPW
CHUNK = 256
NCHUNK = EPW // CHUNK
NP = 10240
ROWS_PER_TILE = NP // 16
ZROWS = 16


def _y_body(x_ref, bf_ref, att_ref, y_ref):
    xb = jnp.dot(x_ref[...], bf_ref[...], preferred_element_type=jnp.float32)
    for r in range(NUM_REL):
        acc = att_ref[r, 0] * xb[:, 0:DP]
        for b in range(1, NB):
            acc = acc + att_ref[r, b] * xb[:, b * DP:(b + 1) * DP]
        y_ref[:, r * DP:(r + 1) * DP] = acc


def _y_matmul(xp, basisflat, att, blk=1000):
    grid = N // blk
    return pl.pallas_call(
        _y_body,
        grid=(grid,),
        in_specs=[
            pl.BlockSpec((blk, DP), lambda i: (i, 0)),
            pl.BlockSpec((DP, NB * DP), lambda i: (0, 0)),
            pl.BlockSpec(memory_space=pltpu.SMEM),
        ],
        out_specs=pl.BlockSpec((blk, NUM_REL * DP), lambda i: (i, 0)),
        out_shape=jax.ShapeDtypeStruct((N, NUM_REL * DP), jnp.float32),
    )(xp, basisflat, att)


def _epi_body(a0_ref, a1_ref, x_ref, root_ref, bias_ref, o_ref, *, relu):
    s = a0_ref[...] + a1_ref[...]
    lanes = lax.broadcasted_iota(jnp.int32, s.shape, 1)
    cnt = jnp.sum(jnp.where(lanes == CNT_COL, s, 0.0), axis=1, keepdims=True)
    denom = jnp.maximum(cnt, 1.0)
    msg = jnp.where(lanes < D, s, 0.0) / denom
    out = msg + jnp.dot(x_ref[...], root_ref[...],
                        preferred_element_type=jnp.float32) + bias_ref[...]
    if relu:
        out = jnp.maximum(out, 0.0)
    o_ref[...] = out


def _epilogue(a0, a1, xp, root_p, bias_p, relu, blk=400):
    grid = N // blk
    return pl.pallas_call(
        functools.partial(_epi_body, relu=relu),
        grid=(grid,),
        in_specs=[
            pl.BlockSpec((blk, DP), lambda i: (i, 0)),
            pl.BlockSpec((blk, DP), lambda i: (i, 0)),
            pl.BlockSpec((blk, DP), lambda i: (i, 0)),
            pl.BlockSpec((DP, DP), lambda i: (0, 0)),
            pl.BlockSpec((1, DP), lambda i: (0, 0)),
        ],
        out_specs=pl.BlockSpec((blk, DP), lambda i: (i, 0)),
        out_shape=jax.ShapeDtypeStruct((N, DP), jnp.float32),
    )(a0, a1, xp, root_p, bias_p)


def _sc_body(y_hbm, gidx_hbm, dst_hbm, nsp_hbm, out_hbm,
             gidx_v, dst_v, nsp_v, rows_v, acc_sh, sem, dsem):
    cid = lax.axis_index("c")
    sid = lax.axis_index("s")
    wid = sid * 2 + cid

    pltpu.sync_copy(gidx_hbm.at[wid], gidx_v)

    zero = jnp.zeros((16,), jnp.float32)
    for q in range(ZROWS):
        for p in range(DP // 16):
            rows_v[q, pl.ds(p * 16, 16)] = zero

    def _zero_step(z, carry):
        pltpu.sync_copy(
            rows_v.at[pl.ds(0, ZROWS)],
            acc_sh.at[pl.ds(sid * ROWS_PER_TILE + z * ZROWS, ZROWS)])
        return carry
    lax.fori_loop(0, ROWS_PER_TILE // ZROWS, _zero_step, 0)
    plsc.subcore_barrier()

    onehot = jnp.where(lax.iota(jnp.int32, 16) == (CNT_COL % 16), 1.0, 0.0)
    cnt_p = CNT_COL // 16

    def _chunk(c, carry):
        desc = pltpu.async_copy(y_hbm.at[gidx_v.at[pl.ds(c * CHUNK, CHUNK)]], rows_v, sem)
        ddesc = pltpu.async_copy(dst_hbm.at[wid, c], dst_v, dsem)
        pltpu.sync_copy(nsp_hbm.at[wid, pl.ds(c * (CHUNK // 8), CHUNK // 8)],
                        nsp_v)
        desc.wait()

        def _grp(g, carry2):
            for k in range(8):
                j = g * 8 + k
                nv = nsp_v[g, pl.ds(k * 16, 16)]
                for p in range(DP // 16):
                    r = rows_v[j, pl.ds(p * 16, 16)] * nv
                    if p == cnt_p:
                        r = r + onehot
                    rows_v[j, pl.ds(p * 16, 16)] = r
            return carry2
        lax.fori_loop(0, CHUNK // 8, _grp, 0)
        ddesc.wait()
        pltpu.sync_copy(rows_v, acc_sh.at[dst_v], add=True)
        return carry
    lax.fori_loop(0, NCHUNK, _chunk, 0)
    plsc.subcore_barrier()

    pltpu.sync_copy(acc_sh.at[pl.ds(sid * ROWS_PER_TILE, ROWS_PER_TILE)],
                    out_hbm.at[cid, pl.ds(sid * ROWS_PER_TILE, ROWS_PER_TILE)])


def _sc_scatter(y, gidx, dst, nsp):
    mesh = plsc.VectorSubcoreMesh(core_axis_name="c", subcore_axis_name="s")
    f = pl.kernel(
        _sc_body,
        out_type=jax.ShapeDtypeStruct((2, NP, DP), jnp.float32),
        mesh=mesh,
        scratch_types=[
            pltpu.VMEM((EPW,), jnp.int32),
            pltpu.VMEM((CHUNK,), jnp.int32),
            pltpu.VMEM((CHUNK // 8, DP), jnp.float32),
            pltpu.VMEM((CHUNK, DP), jnp.float32),
            pltpu.VMEM_SHARED((NP, DP), jnp.float32),
            pltpu.SemaphoreType.DMA,
            pltpu.SemaphoreType.DMA,
        ],
    )
    return f(y.reshape(N * NUM_REL, DP), gidx, dst, nsp)


def _layer(xp, gidx, dst, nsp, basisflat, att, root_p, bias_p, relu):
    y = _y_matmul(xp, basisflat, att)
    acc = _sc_scatter(y, gidx, dst, nsp)
    return _epilogue(acc[0], acc[1], xp, root_p, bias_p, relu)


def kernel(entity, edge_index, edge_attr, edge_type, edge_norm,
           basis1, att1, root1, bias1, We1, be1,
           basis2, att2, root2, bias2, We2, be2):
    del edge_attr, We1, be1, We2, be2

    src = edge_index[0]
    dst = edge_index[1]
    et = edge_type[:, 0]
    pad = EP - E
    gidx = jnp.pad((src * NUM_REL + et).astype(jnp.int32), (0, pad))
    gidx = gidx.reshape(NW, EPW)
    dst_r = jnp.pad(dst.astype(jnp.int32), (0, pad), constant_values=N)
    dst_r = dst_r.reshape(NW, NCHUNK, CHUNK)
    nsp = jnp.pad(edge_norm, (0, pad))
    nsp = jnp.broadcast_to(nsp[:, None], (EP, 16)).reshape(NW, EPW // 8, DP)

    xp = jnp.pad(entity, ((0, 0), (0, DP - D)))

    def prep(basis, root, bias):
        bf = jnp.pad(jnp.transpose(basis, (1, 0, 2)),
                     ((0, DP - D), (0, 0), (0, DP - D))).reshape(DP, NB * DP)
        rp = jnp.pad(root, ((0, DP - D), (0, DP - D)))
        bp = jnp.pad(bias, (0, DP - D)).reshape(1, DP)
        return bf, rp, bp

    bf1, rp1, bp1 = prep(basis1, root1, bias1)
    bf2, rp2, bp2 = prep(basis2, root2, bias2)

    h = _layer(xp, gidx, dst_r, nsp, bf1, att1, rp1, bp1, relu=True)
    out = _layer(h, gidx, dst_r, nsp, bf2, att2, rp2, bp2, relu=False)
    return out[:, :D]


# speedup vs baseline: 1.0666x; 1.0180x over previous
import functools

import jax
import jax.numpy as jnp
from jax import lax
from jax.experimental import pallas as pl
from jax.experimental.pallas import tpu as pltpu
from jax.experimental.pallas import tpu_sc as plsc

N = 10000
E = 160000
D = 100
NUM_REL = 16
NB = 4

DP = 128
CNT_COL = 100
NW = 32
EPW = 5120
EP = NW * EPW
CHUNK = 256
NCHUNK = EPW // CHUNK
NP = 10240
ROWS_PER_TILE = NP // 16
ZROWS = 16


def _y_body(x_ref, bf_ref, att_ref, y_ref):
    xb = jnp.dot(x_ref[...], bf_ref[...], preferred_element_type=jnp.float32)
    for r in range(NUM_REL):
        acc = att_ref[r, 0] * xb[:, 0:DP]
        for b in range(1, NB):
            acc = acc + att_ref[r, b] * xb[:, b * DP:(b + 1) * DP]
        y_ref[:, r * DP:(r + 1) * DP] = acc


def _y_matmul(xp, basisflat, att, blk=1000):
    grid = N // blk
    return pl.pallas_call(
        _y_body,
        grid=(grid,),
        in_specs=[
            pl.BlockSpec((blk, DP), lambda i: (i, 0)),
            pl.BlockSpec((DP, NB * DP), lambda i: (0, 0)),
            pl.BlockSpec(memory_space=pltpu.SMEM),
        ],
        out_specs=pl.BlockSpec((blk, NUM_REL * DP), lambda i: (i, 0)),
        out_shape=jax.ShapeDtypeStruct((N, NUM_REL * DP), jnp.float32),
    )(xp, basisflat, att)


def _epi_body(a0_ref, a1_ref, x_ref, root_ref, bias_ref, o_ref, *, relu):
    s = a0_ref[...] + a1_ref[...]
    lanes = lax.broadcasted_iota(jnp.int32, s.shape, 1)
    cnt = jnp.sum(jnp.where(lanes == CNT_COL, s, 0.0), axis=1, keepdims=True)
    denom = jnp.maximum(cnt, 1.0)
    msg = jnp.where(lanes < D, s, 0.0) / denom
    out = msg + jnp.dot(x_ref[...], root_ref[...],
                        preferred_element_type=jnp.float32) + bias_ref[...]
    if relu:
        out = jnp.maximum(out, 0.0)
    o_ref[...] = out


def _epilogue(a0, a1, xp, root_p, bias_p, relu, blk=1000):
    grid = N // blk
    return pl.pallas_call(
        functools.partial(_epi_body, relu=relu),
        grid=(grid,),
        in_specs=[
            pl.BlockSpec((blk, DP), lambda i: (i, 0)),
            pl.BlockSpec((blk, DP), lambda i: (i, 0)),
            pl.BlockSpec((blk, DP), lambda i: (i, 0)),
            pl.BlockSpec((DP, DP), lambda i: (0, 0)),
            pl.BlockSpec((1, DP), lambda i: (0, 0)),
        ],
        out_specs=pl.BlockSpec((blk, DP), lambda i: (i, 0)),
        out_shape=jax.ShapeDtypeStruct((N, DP), jnp.float32),
    )(a0, a1, xp, root_p, bias_p)


def _sc_body(y_hbm, gidx_hbm, dst_hbm, nsp_hbm, out_hbm,
             gidx_v, dst_v, nsp_v, rows_v, acc_sh, sem, dsem):
    cid = lax.axis_index("c")
    sid = lax.axis_index("s")
    wid = sid * 2 + cid

    pltpu.sync_copy(gidx_hbm.at[wid], gidx_v)

    zero = jnp.zeros((16,), jnp.float32)
    for q in range(ZROWS):
        for p in range(DP // 16):
            rows_v[q, pl.ds(p * 16, 16)] = zero

    def _zero_step(z, carry):
        pltpu.sync_copy(
            rows_v.at[pl.ds(0, ZROWS)],
            acc_sh.at[pl.ds(sid * ROWS_PER_TILE + z * ZROWS, ZROWS)])
        return carry
    lax.fori_loop(0, ROWS_PER_TILE // ZROWS, _zero_step, 0)
    plsc.subcore_barrier()

    onehot = jnp.where(lax.iota(jnp.int32, 16) == (CNT_COL % 16), 1.0, 0.0)
    cnt_p = CNT_COL // 16

    def _chunk(c, carry):
        desc = pltpu.async_copy(y_hbm.at[gidx_v.at[pl.ds(c * CHUNK, CHUNK)]], rows_v, sem)
        ddesc = pltpu.async_copy(dst_hbm.at[wid, c], dst_v, dsem)
        pltpu.sync_copy(nsp_hbm.at[wid, pl.ds(c * (CHUNK // 8), CHUNK // 8)],
                        nsp_v)
        desc.wait()

        def _grp(g, carry2):
            for k in range(8):
                j = g * 8 + k
                nv = nsp_v[g, pl.ds(k * 16, 16)]
                for p in range(DP // 16):
                    r = rows_v[j, pl.ds(p * 16, 16)] * nv
                    if p == cnt_p:
                        r = r + onehot
                    rows_v[j, pl.ds(p * 16, 16)] = r
            return carry2
        lax.fori_loop(0, CHUNK // 8, _grp, 0)
        ddesc.wait()
        pltpu.sync_copy(rows_v, acc_sh.at[dst_v], add=True)
        return carry
    lax.fori_loop(0, NCHUNK, _chunk, 0)
    plsc.subcore_barrier()

    pltpu.sync_copy(acc_sh.at[pl.ds(sid * ROWS_PER_TILE, ROWS_PER_TILE)],
                    out_hbm.at[cid, pl.ds(sid * ROWS_PER_TILE, ROWS_PER_TILE)])


def _sc_scatter(y, gidx, dst, nsp):
    mesh = plsc.VectorSubcoreMesh(core_axis_name="c", subcore_axis_name="s")
    f = pl.kernel(
        _sc_body,
        out_type=jax.ShapeDtypeStruct((2, NP, DP), jnp.float32),
        mesh=mesh,
        scratch_types=[
            pltpu.VMEM((EPW,), jnp.int32),
            pltpu.VMEM((CHUNK,), jnp.int32),
            pltpu.VMEM((CHUNK // 8, DP), jnp.float32),
            pltpu.VMEM((CHUNK, DP), jnp.float32),
            pltpu.VMEM_SHARED((NP, DP), jnp.float32),
            pltpu.SemaphoreType.DMA,
            pltpu.SemaphoreType.DMA,
        ],
    )
    return f(y.reshape(N * NUM_REL, DP), gidx, dst, nsp)


def _layer(xp, gidx, dst, nsp, basisflat, att, root_p, bias_p, relu):
    y = _y_matmul(xp, basisflat, att)
    acc = _sc_scatter(y, gidx, dst, nsp)
    return _epilogue(acc[0], acc[1], xp, root_p, bias_p, relu)


def kernel(entity, edge_index, edge_attr, edge_type, edge_norm,
           basis1, att1, root1, bias1, We1, be1,
           basis2, att2, root2, bias2, We2, be2):
    del edge_attr, We1, be1, We2, be2

    src = edge_index[0]
    dst = edge_index[1]
    et = edge_type[:, 0]
    pad = EP - E
    gidx = jnp.pad((src * NUM_REL + et).astype(jnp.int32), (0, pad))
    gidx = gidx.reshape(NW, EPW)
    dst_r = jnp.pad(dst.astype(jnp.int32), (0, pad), constant_values=N)
    dst_r = dst_r.reshape(NW, NCHUNK, CHUNK)
    nsp = jnp.pad(edge_norm, (0, pad))
    nsp = jnp.broadcast_to(nsp[:, None], (EP, 16)).reshape(NW, EPW // 8, DP)

    xp = jnp.pad(entity, ((0, 0), (0, DP - D)))

    def prep(basis, root, bias):
        bf = jnp.pad(jnp.transpose(basis, (1, 0, 2)),
                     ((0, DP - D), (0, 0), (0, DP - D))).reshape(DP, NB * DP)
        rp = jnp.pad(root, ((0, DP - D), (0, DP - D)))
        bp = jnp.pad(bias, (0, DP - D)).reshape(1, DP)
        return bf, rp, bp

    bf1, rp1, bp1 = prep(basis1, root1, bias1)
    bf2, rp2, bp2 = prep(basis2, root2, bias2)

    h = _layer(xp, gidx, dst_r, nsp, bf1, att1, rp1, bp1, relu=True)
    out = _layer(h, gidx, dst_r, nsp, bf2, att2, rp2, bp2, relu=False)
    return out[:, :D]
